# baseline (speedup 1.0000x reference)
import functools

import jax
import jax.numpy as jnp
from jax import lax
from jax.experimental import pallas as pl
from jax.experimental.pallas import tpu as pltpu
from jax.experimental.pallas import tpu_sc as plsc

_LAMBDA_DIS = 1e-05
_N_NODE = 100000
_EMD_SIZE = 64
_BATCH = 16384

_INFO = plsc.get_sparse_core_info()
_NC = _INFO.num_cores
_NS = _INFO.num_subcores
_L = _INFO.num_lanes
_NW = _NC * _NS
_BPW = _BATCH // _NW
_CHUNK = 128
_NCHUNK = _BPW // _CHUNK


def _bce_l1p(t):
    z = t / (2.0 + t)
    z2 = z * z
    p = 1.0 / 7.0 + z2 * (1.0 / 9.0)
    p = 1.0 / 5.0 + z2 * p
    p = 1.0 / 3.0 + z2 * p
    return 2.0 * z * (1.0 + z2 * p)


@functools.partial(
    pl.kernel,
    mesh=plsc.VectorSubcoreMesh(core_axis_name="c", subcore_axis_name="s"),
    out_type=jax.ShapeDtypeStruct((_NW, _L), jnp.float32),
    scratch_types=[
        pltpu.VMEM((_BPW,), jnp.int32),
        pltpu.VMEM((_BPW,), jnp.int32),
        pltpu.VMEM((_BPW,), jnp.float32),
        pltpu.VMEM((_BPW, _EMD_SIZE), jnp.float32),
        pltpu.VMEM((_BPW, _EMD_SIZE), jnp.float32),
        pltpu.VMEM((_BPW,), jnp.float32),
        pltpu.VMEM((_L,), jnp.float32),
        pltpu.SemaphoreType.DMA,
    ],
    compiler_params=pltpu.CompilerParams(
        needs_layout_passes=False, use_tc_tiling_on_sc=False
    ),
)
def _disc_kernel(node_ids_hbm, neigh_ids_hbm, label_hbm, emd_hbm, bias_hbm,
                 out_hbm, idx_a, idx_b, label_v, a_v, b_v, bias_v, part_v,
                 sem):
    wid = lax.axis_index("s") * _NC + lax.axis_index("c")
    base = wid * _BPW

    pltpu.sync_copy(node_ids_hbm.at[pl.ds(base, _BPW)], idx_a)
    pltpu.sync_copy(neigh_ids_hbm.at[pl.ds(base, _BPW)], idx_b)
    pltpu.sync_copy(label_hbm.at[pl.ds(base, _BPW)], label_v)

    copies = []
    for j in range(_NCHUNK):
        s = pl.ds(j * _CHUNK, _CHUNK)
        copies.append(pltpu.async_copy(emd_hbm.at[idx_a.at[s]], a_v.at[s], sem))
        copies.append(pltpu.async_copy(emd_hbm.at[idx_b.at[s]], b_v.at[s], sem))
        copies.append(pltpu.async_copy(bias_hbm.at[idx_b.at[s]], bias_v.at[s], sem))
    for c in copies:
        c.wait()

    lane = lax.iota(jnp.int32, 16)

    def group_body(g, carry):
        loss_acc, l2_acc = carry
        row0 = g * _L
        score16 = jnp.zeros((_L,), jnp.float32)
        l2s = jnp.zeros((_L,), jnp.float32)
        for r in range(_L):
            i = row0 + r
            s16 = jnp.zeros((_L,), jnp.float32)
            for q in range(_EMD_SIZE // _L):
                va = a_v[i, pl.ds(q * _L, _L)]
                vb = b_v[i, pl.ds(q * _L, _L)]
                s16 = s16 + va * vb
                l2s = l2s + (va * va + vb * vb)
            score16 = score16 + jnp.where(lane == r, jnp.sum(s16), 0.0)
        rows = row0 + lane
        bias16 = plsc.load_gather(bias_v, [rows])
        lab16 = plsc.load_gather(label_v, [rows])
        score16 = score16 + bias16
        l2s = l2s + bias16 * bias16
        t = jnp.exp(-jnp.abs(score16))
        bce = jnp.maximum(score16, 0.0) - score16 * lab16 + _bce_l1p(t)
        return loss_acc + bce, l2_acc + l2s

    zeros = jnp.zeros((_L,), jnp.float32)
    loss_acc, l2_acc = lax.fori_loop(0, _BPW // _L, group_body, (zeros, zeros))

    part_v[...] = loss_acc * (1.0 / _BATCH) + (0.5 * _LAMBDA_DIS) * l2_acc
    pltpu.sync_copy(part_v, out_hbm.at[wid])


def kernel(node_ids, neighbor_ids, label, node_emd, bias_vector):
    parts = _disc_kernel(node_ids, neighbor_ids, label, node_emd, bias_vector)
    return jnp.sum(parts)
